# baseline (speedup 1.0000x reference)
import functools

import jax
import jax.numpy as jnp
from jax import lax
from jax.experimental import pallas as pl
from jax.experimental.pallas import tpu as pltpu
from jax.experimental.pallas import tpu_sc as plsc

VOCAB = 100000
EMBED = 64
BATCH = 4096
SEQ = 50
TOTAL = BATCH * SEQ

NC = 2
NS = 16
NW = NC * NS
RPW = TOTAL // NW
CHUNK = 128
NCH = RPW // CHUNK
K = 5
G = NCH // K

TB = 512
OB = 512

_mesh = plsc.VectorSubcoreMesh(core_axis_name="c", subcore_axis_name="s")


@functools.partial(
    pl.kernel,
    out_type=jax.ShapeDtypeStruct((TOTAL, EMBED), jnp.float32),
    mesh=_mesh,
    compiler_params=pltpu.CompilerParams(use_tc_tiling_on_sc=False),
    scratch_types=[
        pltpu.VMEM((NCH, CHUNK), jnp.int32),
        pltpu.VMEM((K * CHUNK, EMBED), jnp.float32),
        pltpu.VMEM((K * CHUNK, EMBED), jnp.float32),
        pltpu.SemaphoreType.DMA,
        pltpu.SemaphoreType.DMA,
        pltpu.SemaphoreType.DMA,
        pltpu.SemaphoreType.DMA,
    ],
)
def _sc_gather(idx_hbm, tab_hbm, out_hbm,
               idx_v, set_a, set_b, gs_a, gs_b, ws_a, ws_b):
    wid = lax.axis_index("s") * NC + lax.axis_index("c")
    base = wid * RPW
    pltpu.sync_copy(idx_hbm.at[wid], idx_v)

    def issue_gathers(g, dst, gsem):
        for i in range(K):
            pltpu.async_copy(tab_hbm.at[idx_v.at[g * K + i]],
                             dst.at[pl.ds(i * CHUNK, CHUNK)], gsem)

    def drain_gathers(dst, gsem):
        for i in range(K):
            pltpu.make_async_copy(
                tab_hbm.at[pl.ds(0, CHUNK)],
                dst.at[pl.ds(i * CHUNK, CHUNK)], gsem).wait()

    def issue_writeback(g, src, wsem):
        pltpu.async_copy(
            src, out_hbm.at[pl.ds(base + g * K * CHUNK, K * CHUNK)], wsem)

    def wait_writeback(src, wsem):
        pltpu.make_async_copy(
            src, out_hbm.at[pl.ds(base, K * CHUNK)], wsem).wait()

    issue_gathers(0, set_a, gs_a)

    def body(t, carry):
        @pl.when((t % 2) == 0)
        def _():
            @pl.when(t >= 1)
            def _():
                wait_writeback(set_b, ws_b)
            issue_gathers(t + 1, set_b, gs_b)
            drain_gathers(set_a, gs_a)
            issue_writeback(t, set_a, ws_a)

        @pl.when((t % 2) == 1)
        def _():
            wait_writeback(set_a, ws_a)
            issue_gathers(t + 1, set_a, gs_a)
            drain_gathers(set_b, gs_b)
            issue_writeback(t, set_b, ws_b)

        return carry

    lax.fori_loop(0, G - 1, body, 0)

    drain_gathers(set_b, gs_b)
    issue_writeback(G - 1, set_b, ws_b)
    wait_writeback(set_a, ws_a)
    wait_writeback(set_b, ws_b)


def _tc_transpose_body(in_ref, out_ref):
    out_ref[...] = in_ref[...].T


_tc_table_transpose = pl.pallas_call(
    _tc_transpose_body,
    grid=(pl.cdiv(VOCAB, TB),),
    in_specs=[pl.BlockSpec((EMBED, TB), lambda i: (0, i))],
    out_specs=pl.BlockSpec((TB, EMBED), lambda i: (i, 0)),
    out_shape=jax.ShapeDtypeStruct((VOCAB, EMBED), jnp.float32),
)


_tc_out_transpose = pl.pallas_call(
    _tc_transpose_body,
    grid=(SEQ, BATCH // OB),
    in_specs=[pl.BlockSpec((OB, EMBED),
                           lambda s, i: (s * (BATCH // OB) + i, 0))],
    out_specs=pl.BlockSpec((EMBED, OB), lambda s, i: (s, i)),
    out_shape=jax.ShapeDtypeStruct((SEQ * EMBED, BATCH), jnp.float32),
)


def kernel(src_indices, tgt_indices, source_table, target_table):
    si = jnp.transpose(src_indices).reshape(NW, NCH, CHUNK)
    ti = jnp.transpose(tgt_indices).reshape(NW, NCH, CHUNK)
    st = _tc_table_transpose(jnp.transpose(source_table))
    tt = _tc_table_transpose(jnp.transpose(target_table))
    so = _sc_gather(si, st)
    to = _sc_gather(ti, tt)

    def to_entry(y):
        z = _tc_out_transpose(y)
        return jnp.transpose(z.reshape(SEQ, EMBED, BATCH), (2, 0, 1))

    return (to_entry(so), to_entry(to))


# speedup vs baseline: 1.7449x; 1.7449x over previous
import functools

import jax
import jax.numpy as jnp
from jax import lax
from jax.experimental import pallas as pl
from jax.experimental.pallas import tpu as pltpu
from jax.experimental.pallas import tpu_sc as plsc

VOCAB = 100000
EMBED = 64
BATCH = 4096
SEQ = 50
TOTAL = BATCH * SEQ

NC = 2
NS = 16
NW = NC * NS
RPW = TOTAL // NW
CHUNK = 128
NCH = RPW // CHUNK
K = 5
G = NCH // K

TB = 2048
OB = 2048

_mesh = plsc.VectorSubcoreMesh(core_axis_name="c", subcore_axis_name="s")


@functools.partial(
    pl.kernel,
    out_type=jax.ShapeDtypeStruct((TOTAL, EMBED), jnp.float32),
    mesh=_mesh,
    compiler_params=pltpu.CompilerParams(use_tc_tiling_on_sc=False),
    scratch_types=[
        pltpu.VMEM((NCH, CHUNK), jnp.int32),
        pltpu.VMEM((K * CHUNK, EMBED), jnp.float32),
        pltpu.VMEM((K * CHUNK, EMBED), jnp.float32),
        pltpu.SemaphoreType.DMA,
        pltpu.SemaphoreType.DMA,
        pltpu.SemaphoreType.DMA,
        pltpu.SemaphoreType.DMA,
    ],
)
def _sc_gather(idx_hbm, tab_hbm, out_hbm,
               idx_v, set_a, set_b, gs_a, gs_b, ws_a, ws_b):
    wid = lax.axis_index("s") * NC + lax.axis_index("c")
    base = wid * RPW
    pltpu.sync_copy(idx_hbm.at[wid], idx_v)

    def issue_gathers(g, dst, gsem):
        for i in range(K):
            pltpu.async_copy(tab_hbm.at[idx_v.at[g * K + i]],
                             dst.at[pl.ds(i * CHUNK, CHUNK)], gsem)

    def drain_gathers(dst, gsem):
        for i in range(K):
            pltpu.make_async_copy(
                tab_hbm.at[pl.ds(0, CHUNK)],
                dst.at[pl.ds(i * CHUNK, CHUNK)], gsem).wait()

    def issue_writeback(g, src, wsem):
        pltpu.async_copy(
            src, out_hbm.at[pl.ds(base + g * K * CHUNK, K * CHUNK)], wsem)

    def wait_writeback(src, wsem):
        pltpu.make_async_copy(
            src, out_hbm.at[pl.ds(base, K * CHUNK)], wsem).wait()

    issue_gathers(0, set_a, gs_a)

    def body(t, carry):
        @pl.when((t % 2) == 0)
        def _():
            @pl.when(t >= 1)
            def _():
                wait_writeback(set_b, ws_b)
            issue_gathers(t + 1, set_b, gs_b)
            drain_gathers(set_a, gs_a)
            issue_writeback(t, set_a, ws_a)

        @pl.when((t % 2) == 1)
        def _():
            wait_writeback(set_a, ws_a)
            issue_gathers(t + 1, set_a, gs_a)
            drain_gathers(set_b, gs_b)
            issue_writeback(t, set_b, ws_b)

        return carry

    lax.fori_loop(0, G - 1, body, 0)

    drain_gathers(set_b, gs_b)
    issue_writeback(G - 1, set_b, ws_b)
    wait_writeback(set_a, ws_a)
    wait_writeback(set_b, ws_b)


def _eye64():
    r = lax.broadcasted_iota(jnp.int32, (EMBED, EMBED), 0)
    c = lax.broadcasted_iota(jnp.int32, (EMBED, EMBED), 1)
    return (r == c).astype(jnp.float32)


def _tc_table_transpose_body(in_ref, out_ref):
    out_ref[...] = lax.dot_general(
        in_ref[...], _eye64(), (((0,), (0,)), ((), ())),
        preferred_element_type=jnp.float32)


def _tc_out_transpose_body(in_ref, out_ref):
    out_ref[...] = lax.dot_general(
        _eye64(), in_ref[...], (((1,), (1,)), ((), ())),
        preferred_element_type=jnp.float32)


_tc_table_transpose = pl.pallas_call(
    _tc_table_transpose_body,
    grid=(pl.cdiv(VOCAB, TB),),
    in_specs=[pl.BlockSpec((EMBED, TB), lambda i: (0, i))],
    out_specs=pl.BlockSpec((TB, EMBED), lambda i: (i, 0)),
    out_shape=jax.ShapeDtypeStruct((VOCAB, EMBED), jnp.float32),
)


_tc_out_transpose = pl.pallas_call(
    _tc_out_transpose_body,
    grid=(SEQ, BATCH // OB),
    in_specs=[pl.BlockSpec((OB, EMBED),
                           lambda s, i: (s * (BATCH // OB) + i, 0))],
    out_specs=pl.BlockSpec((EMBED, OB), lambda s, i: (s, i)),
    out_shape=jax.ShapeDtypeStruct((SEQ * EMBED, BATCH), jnp.float32),
)


def kernel(src_indices, tgt_indices, source_table, target_table):
    si = jnp.transpose(src_indices).reshape(NW, NCH, CHUNK)
    ti = jnp.transpose(tgt_indices).reshape(NW, NCH, CHUNK)
    st = _tc_table_transpose(jnp.transpose(source_table))
    tt = _tc_table_transpose(jnp.transpose(target_table))
    so = _sc_gather(si, st)
    to = _sc_gather(ti, tt)

    def to_entry(y):
        z = _tc_out_transpose(y)
        return jnp.transpose(z.reshape(SEQ, EMBED, BATCH), (2, 0, 1))

    return (to_entry(so), to_entry(to))


# speedup vs baseline: 2.0482x; 1.1738x over previous
import functools

import jax
import jax.numpy as jnp
from jax import lax
from jax.experimental import pallas as pl
from jax.experimental.pallas import tpu as pltpu
from jax.experimental.pallas import tpu_sc as plsc

VOCAB = 100000
EMBED = 64
BATCH = 4096
SEQ = 50
TOTAL = BATCH * SEQ

NC = 2
NS = 16
NW = NC * NS
RPW = TOTAL // NW
CHUNK = 128
NCH = RPW // CHUNK
K = 5
G = NCH // K

TB = 8192
OB = 4096

_mesh = plsc.VectorSubcoreMesh(core_axis_name="c", subcore_axis_name="s")


@functools.partial(
    pl.kernel,
    out_type=jax.ShapeDtypeStruct((TOTAL, EMBED), jnp.float32),
    mesh=_mesh,
    compiler_params=pltpu.CompilerParams(use_tc_tiling_on_sc=False),
    scratch_types=[
        pltpu.VMEM((NCH, CHUNK), jnp.int32),
        pltpu.VMEM((K * CHUNK, EMBED), jnp.float32),
        pltpu.VMEM((K * CHUNK, EMBED), jnp.float32),
        pltpu.SemaphoreType.DMA,
        pltpu.SemaphoreType.DMA,
        pltpu.SemaphoreType.DMA,
        pltpu.SemaphoreType.DMA,
    ],
)
def _sc_gather(idx_hbm, tab_hbm, out_hbm,
               idx_v, set_a, set_b, gs_a, gs_b, ws_a, ws_b):
    wid = lax.axis_index("s") * NC + lax.axis_index("c")
    base = wid * RPW
    pltpu.sync_copy(idx_hbm.at[wid], idx_v)

    def issue_gathers(g, dst, gsem):
        for i in range(K):
            pltpu.async_copy(tab_hbm.at[idx_v.at[g * K + i]],
                             dst.at[pl.ds(i * CHUNK, CHUNK)], gsem)

    def drain_gathers(dst, gsem):
        for i in range(K):
            pltpu.make_async_copy(
                tab_hbm.at[pl.ds(0, CHUNK)],
                dst.at[pl.ds(i * CHUNK, CHUNK)], gsem).wait()

    def issue_writeback(g, src, wsem):
        pltpu.async_copy(
            src, out_hbm.at[pl.ds(base + g * K * CHUNK, K * CHUNK)], wsem)

    def wait_writeback(src, wsem):
        pltpu.make_async_copy(
            src, out_hbm.at[pl.ds(base, K * CHUNK)], wsem).wait()

    issue_gathers(0, set_a, gs_a)

    def body(t, carry):
        @pl.when((t % 2) == 0)
        def _():
            @pl.when(t >= 1)
            def _():
                wait_writeback(set_b, ws_b)
            issue_gathers(t + 1, set_b, gs_b)
            drain_gathers(set_a, gs_a)
            issue_writeback(t, set_a, ws_a)

        @pl.when((t % 2) == 1)
        def _():
            wait_writeback(set_a, ws_a)
            issue_gathers(t + 1, set_a, gs_a)
            drain_gathers(set_b, gs_b)
            issue_writeback(t, set_b, ws_b)

        return carry

    lax.fori_loop(0, G - 1, body, 0)

    drain_gathers(set_b, gs_b)
    issue_writeback(G - 1, set_b, ws_b)
    wait_writeback(set_a, ws_a)
    wait_writeback(set_b, ws_b)


def _eye64():
    r = lax.broadcasted_iota(jnp.int32, (EMBED, EMBED), 0)
    c = lax.broadcasted_iota(jnp.int32, (EMBED, EMBED), 1)
    return (r == c).astype(jnp.float32)


def _tc_table_transpose_body(in_ref, out_ref):
    out_ref[...] = lax.dot_general(
        in_ref[...], _eye64(), (((0,), (0,)), ((), ())),
        preferred_element_type=jnp.float32)


def _tc_out_transpose_body(in_ref, out_ref):
    out_ref[...] = lax.dot_general(
        _eye64(), in_ref[...], (((1,), (1,)), ((), ())),
        preferred_element_type=jnp.float32)


_tc_table_transpose = pl.pallas_call(
    _tc_table_transpose_body,
    grid=(pl.cdiv(VOCAB, TB),),
    in_specs=[pl.BlockSpec((EMBED, TB), lambda i: (0, i))],
    out_specs=pl.BlockSpec((TB, EMBED), lambda i: (i, 0)),
    out_shape=jax.ShapeDtypeStruct((VOCAB, EMBED), jnp.float32),
)


_tc_out_transpose = pl.pallas_call(
    _tc_out_transpose_body,
    grid=(SEQ,),
    in_specs=[pl.BlockSpec((OB, EMBED), lambda s: (s, 0))],
    out_specs=pl.BlockSpec((EMBED, OB), lambda s: (s, 0)),
    out_shape=jax.ShapeDtypeStruct((SEQ * EMBED, BATCH), jnp.float32),
)


def kernel(src_indices, tgt_indices, source_table, target_table):
    si = jnp.transpose(src_indices).reshape(NW, NCH, CHUNK)
    ti = jnp.transpose(tgt_indices).reshape(NW, NCH, CHUNK)
    st = _tc_table_transpose(jnp.transpose(source_table))
    tt = _tc_table_transpose(jnp.transpose(target_table))
    so = _sc_gather(si, st)
    to = _sc_gather(ti, tt)

    def to_entry(y):
        z = _tc_out_transpose(y)
        return jnp.transpose(z.reshape(SEQ, EMBED, BATCH), (2, 0, 1))

    return (to_entry(so), to_entry(to))


# speedup vs baseline: 2.6534x; 1.2955x over previous
import functools

import jax
import jax.numpy as jnp
from jax import lax
from jax.experimental import pallas as pl
from jax.experimental.pallas import tpu as pltpu
from jax.experimental.pallas import tpu_sc as plsc

VOCAB = 100000
EMBED = 64
BATCH = 4096
SEQ = 50
TOTAL = BATCH * SEQ
HALF = BATCH // 2

NC = 2
NS = 16
NW = NC * NS
RPW = TOTAL // NW
CHUNK = 128
NCH = RPW // CHUNK
K = 5
G = NCH // K

OS = 2

_mesh = plsc.VectorSubcoreMesh(core_axis_name="c", subcore_axis_name="s")


@functools.partial(
    pl.kernel,
    out_type=jax.ShapeDtypeStruct((TOTAL, EMBED), jnp.float32),
    mesh=_mesh,
    compiler_params=pltpu.CompilerParams(use_tc_tiling_on_sc=False),
    scratch_types=[
        pltpu.VMEM((NCH, CHUNK), jnp.int32),
        pltpu.VMEM((K * CHUNK, EMBED), jnp.float32),
        pltpu.VMEM((K * CHUNK, EMBED), jnp.float32),
        pltpu.SemaphoreType.DMA,
        pltpu.SemaphoreType.DMA,
        pltpu.SemaphoreType.DMA,
        pltpu.SemaphoreType.DMA,
    ],
)
def _sc_gather(idx_hbm, tab_hbm, out_hbm,
               idx_v, set_a, set_b, gs_a, gs_b, ws_a, ws_b):
    wid = lax.axis_index("s") * NC + lax.axis_index("c")
    base = wid * RPW
    pltpu.sync_copy(idx_hbm.at[wid], idx_v)

    def issue_gathers(g, dst, gsem):
        for i in range(K):
            pltpu.async_copy(tab_hbm.at[idx_v.at[g * K + i]],
                             dst.at[pl.ds(i * CHUNK, CHUNK)], gsem)

    def drain_gathers(dst, gsem):
        for i in range(K):
            pltpu.make_async_copy(
                tab_hbm.at[pl.ds(0, CHUNK)],
                dst.at[pl.ds(i * CHUNK, CHUNK)], gsem).wait()

    def issue_writeback(g, src, wsem):
        pltpu.async_copy(
            src, out_hbm.at[pl.ds(base + g * K * CHUNK, K * CHUNK)], wsem)

    def wait_writeback(src, wsem):
        pltpu.make_async_copy(
            src, out_hbm.at[pl.ds(base, K * CHUNK)], wsem).wait()

    issue_gathers(0, set_a, gs_a)

    def body(t, carry):
        @pl.when((t % 2) == 0)
        def _():
            @pl.when(t >= 1)
            def _():
                wait_writeback(set_b, ws_b)
            issue_gathers(t + 1, set_b, gs_b)
            drain_gathers(set_a, gs_a)
            issue_writeback(t, set_a, ws_a)

        @pl.when((t % 2) == 1)
        def _():
            wait_writeback(set_a, ws_a)
            issue_gathers(t + 1, set_a, gs_a)
            drain_gathers(set_b, gs_b)
            issue_writeback(t, set_b, ws_b)

        return carry

    lax.fori_loop(0, G - 1, body, 0)

    drain_gathers(set_b, gs_b)
    issue_writeback(G - 1, set_b, ws_b)
    wait_writeback(set_a, ws_a)
    wait_writeback(set_b, ws_b)


def _eye64():
    r = lax.broadcasted_iota(jnp.int32, (EMBED, EMBED), 0)
    c = lax.broadcasted_iota(jnp.int32, (EMBED, EMBED), 1)
    return (r == c).astype(jnp.float32)


def _tc_out_body(in_ref, out_ref):
    eye = _eye64()
    for s in range(OS):
        blk = in_ref[pl.ds(s * HALF, HALF), :]
        lo = lax.dot_general(eye, blk[:, :EMBED],
                             (((1,), (1,)), ((), ())),
                             preferred_element_type=jnp.float32)
        hi = lax.dot_general(eye, blk[:, EMBED:],
                             (((1,), (1,)), ((), ())),
                             preferred_element_type=jnp.float32)
        out_ref[pl.ds(s * EMBED, EMBED), :] = jnp.concatenate(
            [lo, hi], axis=1)


_tc_out_transpose = pl.pallas_call(
    _tc_out_body,
    grid=(SEQ // OS,),
    in_specs=[pl.BlockSpec((OS * HALF, 2 * EMBED), lambda i: (i, 0))],
    out_specs=pl.BlockSpec((OS * EMBED, BATCH), lambda i: (i, 0)),
    out_shape=jax.ShapeDtypeStruct((SEQ * EMBED, BATCH), jnp.float32),
)


def _permute_indices(idx):
    t = jnp.transpose(idx)
    t = jnp.swapaxes(t.reshape(SEQ, 2, HALF), 1, 2)
    return t.reshape(NW, NCH, CHUNK)


def kernel(src_indices, tgt_indices, source_table, target_table):
    si = _permute_indices(src_indices)
    ti = _permute_indices(tgt_indices)
    so = _sc_gather(si, source_table)
    to = _sc_gather(ti, target_table)

    def to_entry(y):
        z = _tc_out_transpose(y.reshape(TOTAL // 2, 2 * EMBED))
        return jnp.transpose(z.reshape(SEQ, EMBED, BATCH), (2, 0, 1))

    return (to_entry(so), to_entry(to))


# speedup vs baseline: 4.6241x; 1.7427x over previous
import functools

import jax
import jax.numpy as jnp
from jax import lax
from jax.experimental import pallas as pl
from jax.experimental.pallas import tpu as pltpu
from jax.experimental.pallas import tpu_sc as plsc

VOCAB = 100000
EMBED = 64
BATCH = 4096
SEQ = 50
TOTAL = BATCH * SEQ
HALF = BATCH // 2

SPLIT = 51200
PACKED = SPLIT

NC = 2
NS = 16
NW = NC * NS
RPW = TOTAL // NW
CHUNK = 128
NCH = RPW // CHUNK
K = 5
G = NCH // K
NPLANE = 3

TV = 2048
OS = 2

_mesh = plsc.VectorSubcoreMesh(core_axis_name="c", subcore_axis_name="s")


@functools.partial(
    pl.kernel,
    out_type=jax.ShapeDtypeStruct((TOTAL, EMBED), jnp.float32),
    mesh=_mesh,
    compiler_params=pltpu.CompilerParams(use_tc_tiling_on_sc=False,
                                         needs_layout_passes=False),
    scratch_types=[
        pltpu.VMEM((NPLANE * BATCH,), jnp.int32),
        pltpu.VMEM((NCH, CHUNK), jnp.int32),
        pltpu.VMEM((K * CHUNK, EMBED), jnp.float32),
        pltpu.VMEM((K * CHUNK, EMBED), jnp.float32),
        pltpu.SemaphoreType.DMA,
        pltpu.SemaphoreType.DMA,
        pltpu.SemaphoreType.DMA,
        pltpu.SemaphoreType.DMA,
    ],
)
def _sc_gather(idx_hbm, tab_hbm, out_hbm,
               plane_v, idx_v, set_a, set_b, gs_a, gs_b, ws_a, ws_b):
    wid = lax.axis_index("s") * NC + lax.axis_index("c")
    base = wid * RPW
    s_lo = jnp.minimum(base // BATCH, SEQ - NPLANE)
    for k in range(NPLANE):
        pltpu.sync_copy(idx_hbm.at[s_lo + k],
                        plane_v.at[pl.ds(k * BATCH, BATCH)])

    def remap(a):
        a2 = a * 2
        return jnp.where(a < SPLIT, a2, a2 - (2 * SPLIT - 1))

    lanes = lax.iota(jnp.int32, 16)

    def build(c, carry):
        j0 = base + c * CHUNK
        s_off = j0 // BATCH - s_lo
        q0 = (j0 % BATCH) // 2
        rows = jnp.full((16,), c, jnp.int32)
        for i in range(4):
            off = s_off * BATCH + q0 + 16 * i
            a = plane_v[pl.ds(off, 16)]
            b = plane_v[pl.ds(off + HALF, 16)]
            cols = lanes * 2 + 32 * i
            plsc.store_scatter(idx_v, [rows, cols], remap(a))
            plsc.store_scatter(idx_v, [rows, cols + 1], remap(b))
        return carry

    lax.fori_loop(0, NCH, build, 0)

    def issue_gathers(g, dst, gsem):
        for i in range(K):
            pltpu.async_copy(tab_hbm.at[idx_v.at[g * K + i]],
                             dst.at[pl.ds(i * CHUNK, CHUNK)], gsem)

    def drain_gathers(dst, gsem):
        for i in range(K):
            pltpu.make_async_copy(
                tab_hbm.at[pl.ds(0, CHUNK)],
                dst.at[pl.ds(i * CHUNK, CHUNK)], gsem).wait()

    def issue_writeback(g, src, wsem):
        pltpu.async_copy(
            src, out_hbm.at[pl.ds(base + g * K * CHUNK, K * CHUNK)], wsem)

    def wait_writeback(src, wsem):
        pltpu.make_async_copy(
            src, out_hbm.at[pl.ds(base, K * CHUNK)], wsem).wait()

    issue_gathers(0, set_a, gs_a)

    def body(t, carry):
        @pl.when((t % 2) == 0)
        def _():
            @pl.when(t >= 1)
            def _():
                wait_writeback(set_b, ws_b)
            issue_gathers(t + 1, set_b, gs_b)
            drain_gathers(set_a, gs_a)
            issue_writeback(t, set_a, ws_a)

        @pl.when((t % 2) == 1)
        def _():
            wait_writeback(set_a, ws_a)
            issue_gathers(t + 1, set_a, gs_a)
            drain_gathers(set_b, gs_b)
            issue_writeback(t, set_b, ws_b)

        return carry

    lax.fori_loop(0, G - 1, body, 0)

    drain_gathers(set_b, gs_b)
    issue_writeback(G - 1, set_b, ws_b)
    wait_writeback(set_a, ws_a)
    wait_writeback(set_b, ws_b)


def _eye64():
    r = lax.broadcasted_iota(jnp.int32, (EMBED, EMBED), 0)
    c = lax.broadcasted_iota(jnp.int32, (EMBED, EMBED), 1)
    return (r == c).astype(jnp.float32)


def _tc_pack_body(inl_ref, inr_ref, out_ref):
    eye = _eye64()
    lo = lax.dot_general(inl_ref[...], eye, (((0,), (0,)), ((), ())),
                         preferred_element_type=jnp.float32)
    hi = lax.dot_general(inr_ref[...], eye, (((0,), (0,)), ((), ())),
                         preferred_element_type=jnp.float32)
    out_ref[...] = jnp.concatenate([lo, hi], axis=1)


_NB_L = SPLIT // TV


_tc_table_pack = pl.pallas_call(
    _tc_pack_body,
    grid=(_NB_L,),
    in_specs=[
        pl.BlockSpec((EMBED, TV), lambda i: (0, i)),
        pl.BlockSpec((EMBED, TV),
                     lambda i: (0, jnp.minimum(_NB_L + i,
                                               (VOCAB - 1) // TV))),
    ],
    out_specs=pl.BlockSpec((TV, 2 * EMBED), lambda i: (i, 0)),
    out_shape=jax.ShapeDtypeStruct((PACKED, 2 * EMBED), jnp.float32),
)


def _tc_out_body(in_ref, out_ref):
    eye = _eye64()
    for s in range(OS):
        blk = in_ref[pl.ds(s * HALF, HALF), :]
        lo = lax.dot_general(eye, blk[:, :EMBED],
                             (((1,), (1,)), ((), ())),
                             preferred_element_type=jnp.float32)
        hi = lax.dot_general(eye, blk[:, EMBED:],
                             (((1,), (1,)), ((), ())),
                             preferred_element_type=jnp.float32)
        out_ref[pl.ds(s * EMBED, EMBED), :] = jnp.concatenate(
            [lo, hi], axis=1)


_tc_out_transpose = pl.pallas_call(
    _tc_out_body,
    grid=(SEQ // OS,),
    in_specs=[pl.BlockSpec((OS * HALF, 2 * EMBED), lambda i: (i, 0))],
    out_specs=pl.BlockSpec((OS * EMBED, BATCH), lambda i: (i, 0)),
    out_shape=jax.ShapeDtypeStruct((SEQ * EMBED, BATCH), jnp.float32),
)


def kernel(src_indices, tgt_indices, source_table, target_table):
    st = _tc_table_pack(jnp.transpose(source_table),
                        jnp.transpose(source_table))
    tt = _tc_table_pack(jnp.transpose(target_table),
                        jnp.transpose(target_table))
    so = _sc_gather(jnp.transpose(src_indices), st.reshape(2 * PACKED, EMBED))
    to = _sc_gather(jnp.transpose(tgt_indices), tt.reshape(2 * PACKED, EMBED))

    def to_entry(y):
        z = _tc_out_transpose(y.reshape(TOTAL // 2, 2 * EMBED))
        return jnp.transpose(z.reshape(SEQ, EMBED, BATCH), (2, 0, 1))

    return (to_entry(so), to_entry(to))


# speedup vs baseline: 4.7506x; 1.0274x over previous
import functools

import jax
import jax.numpy as jnp
from jax import lax
from jax.experimental import pallas as pl
from jax.experimental.pallas import tpu as pltpu
from jax.experimental.pallas import tpu_sc as plsc

VOCAB = 100000
EMBED = 64
BATCH = 4096
SEQ = 50
TOTAL = BATCH * SEQ
HALF = BATCH // 2

SPLIT = 51200
PACKED = SPLIT

NC = 2
NS = 16
NW = NC * NS
RPW = TOTAL // NW
CHUNK = 128
NCH = RPW // CHUNK
K = 5
G = NCH // K
NPLANE = 3

TV = 2048
OS = 5

_mesh = plsc.VectorSubcoreMesh(core_axis_name="c", subcore_axis_name="s")


@functools.partial(
    pl.kernel,
    out_type=jax.ShapeDtypeStruct((TOTAL, EMBED), jnp.float32),
    mesh=_mesh,
    compiler_params=pltpu.CompilerParams(use_tc_tiling_on_sc=False,
                                         needs_layout_passes=False),
    scratch_types=[
        pltpu.VMEM((NPLANE * BATCH,), jnp.int32),
        pltpu.VMEM((NCH, CHUNK), jnp.int32),
        pltpu.VMEM((K * CHUNK, EMBED), jnp.float32),
        pltpu.VMEM((K * CHUNK, EMBED), jnp.float32),
        pltpu.SemaphoreType.DMA,
        pltpu.SemaphoreType.DMA,
        pltpu.SemaphoreType.DMA,
        pltpu.SemaphoreType.DMA,
    ],
)
def _sc_gather(idx_hbm, tab_hbm, out_hbm,
               plane_v, idx_v, set_a, set_b, gs_a, gs_b, ws_a, ws_b):
    wid = lax.axis_index("s") * NC + lax.axis_index("c")
    base = wid * RPW
    s_lo = jnp.minimum(base // BATCH, SEQ - NPLANE)
    for k in range(NPLANE):
        pltpu.sync_copy(idx_hbm.at[s_lo + k],
                        plane_v.at[pl.ds(k * BATCH, BATCH)])

    def remap(a):
        a2 = a * 2
        return jnp.where(a < SPLIT, a2, a2 - (2 * SPLIT - 1))

    lanes = lax.iota(jnp.int32, 16)

    def build(c, carry):
        j0 = base + c * CHUNK
        s_off = j0 // BATCH - s_lo
        q0 = (j0 % BATCH) // 2
        rows = jnp.full((16,), c, jnp.int32)
        for i in range(4):
            off = s_off * BATCH + q0 + 16 * i
            a = plane_v[pl.ds(off, 16)]
            b = plane_v[pl.ds(off + HALF, 16)]
            cols = lanes * 2 + 32 * i
            plsc.store_scatter(idx_v, [rows, cols], remap(a))
            plsc.store_scatter(idx_v, [rows, cols + 1], remap(b))
        return carry

    lax.fori_loop(0, NCH, build, 0)

    def issue_gathers(g, dst, gsem):
        for i in range(K):
            pltpu.async_copy(tab_hbm.at[idx_v.at[g * K + i]],
                             dst.at[pl.ds(i * CHUNK, CHUNK)], gsem)

    def drain_gathers(dst, gsem):
        for i in range(K):
            pltpu.make_async_copy(
                tab_hbm.at[pl.ds(0, CHUNK)],
                dst.at[pl.ds(i * CHUNK, CHUNK)], gsem).wait()

    def issue_writeback(g, src, wsem):
        pltpu.async_copy(
            src, out_hbm.at[pl.ds(base + g * K * CHUNK, K * CHUNK)], wsem)

    def wait_writeback(src, wsem):
        pltpu.make_async_copy(
            src, out_hbm.at[pl.ds(base, K * CHUNK)], wsem).wait()

    issue_gathers(0, set_a, gs_a)

    def body(t, carry):
        @pl.when((t % 2) == 0)
        def _():
            @pl.when(t >= 1)
            def _():
                wait_writeback(set_b, ws_b)
            issue_gathers(t + 1, set_b, gs_b)
            drain_gathers(set_a, gs_a)
            issue_writeback(t, set_a, ws_a)

        @pl.when((t % 2) == 1)
        def _():
            wait_writeback(set_a, ws_a)
            issue_gathers(t + 1, set_a, gs_a)
            drain_gathers(set_b, gs_b)
            issue_writeback(t, set_b, ws_b)

        return carry

    lax.fori_loop(0, G - 1, body, 0)

    drain_gathers(set_b, gs_b)
    issue_writeback(G - 1, set_b, ws_b)
    wait_writeback(set_a, ws_a)
    wait_writeback(set_b, ws_b)


def _eye64():
    r = lax.broadcasted_iota(jnp.int32, (EMBED, EMBED), 0)
    c = lax.broadcasted_iota(jnp.int32, (EMBED, EMBED), 1)
    return (r == c).astype(jnp.float32)


def _tc_pack_body(inl_ref, inr_ref, out_ref):
    eye = _eye64()
    lo = lax.dot_general(inl_ref[...], eye, (((0,), (0,)), ((), ())),
                         preferred_element_type=jnp.float32)
    hi = lax.dot_general(inr_ref[...], eye, (((0,), (0,)), ((), ())),
                         preferred_element_type=jnp.float32)
    out_ref[...] = jnp.concatenate([lo, hi], axis=1)


_NB_L = SPLIT // TV


_tc_table_pack = pl.pallas_call(
    _tc_pack_body,
    grid=(_NB_L,),
    in_specs=[
        pl.BlockSpec((EMBED, TV), lambda i: (0, i)),
        pl.BlockSpec((EMBED, TV),
                     lambda i: (0, jnp.minimum(_NB_L + i,
                                               (VOCAB - 1) // TV))),
    ],
    out_specs=pl.BlockSpec((TV, 2 * EMBED), lambda i: (i, 0)),
    out_shape=jax.ShapeDtypeStruct((PACKED, 2 * EMBED), jnp.float32),
)


def _tc_out_body(in_ref, out_ref):
    eye = _eye64()
    for s in range(OS):
        blk = in_ref[pl.ds(s * HALF, HALF), :]
        lo = lax.dot_general(eye, blk[:, :EMBED],
                             (((1,), (1,)), ((), ())),
                             preferred_element_type=jnp.float32)
        hi = lax.dot_general(eye, blk[:, EMBED:],
                             (((1,), (1,)), ((), ())),
                             preferred_element_type=jnp.float32)
        out_ref[pl.ds(s * EMBED, EMBED), :] = jnp.concatenate(
            [lo, hi], axis=1)


_tc_out_transpose = pl.pallas_call(
    _tc_out_body,
    grid=(SEQ // OS,),
    in_specs=[pl.BlockSpec((OS * HALF, 2 * EMBED), lambda i: (i, 0))],
    out_specs=pl.BlockSpec((OS * EMBED, BATCH), lambda i: (i, 0)),
    out_shape=jax.ShapeDtypeStruct((SEQ * EMBED, BATCH), jnp.float32),
)


def kernel(src_indices, tgt_indices, source_table, target_table):
    st = _tc_table_pack(jnp.transpose(source_table),
                        jnp.transpose(source_table))
    tt = _tc_table_pack(jnp.transpose(target_table),
                        jnp.transpose(target_table))
    so = _sc_gather(jnp.transpose(src_indices), st.reshape(2 * PACKED, EMBED))
    to = _sc_gather(jnp.transpose(tgt_indices), tt.reshape(2 * PACKED, EMBED))

    def to_entry(y):
        z = _tc_out_transpose(y.reshape(TOTAL // 2, 2 * EMBED))
        return jnp.transpose(z.reshape(SEQ, EMBED, BATCH), (2, 0, 1))

    return (to_entry(so), to_entry(to))


# speedup vs baseline: 4.7788x; 1.0059x over previous
import functools

import jax
import jax.numpy as jnp
from jax import lax
from jax.experimental import pallas as pl
from jax.experimental.pallas import tpu as pltpu
from jax.experimental.pallas import tpu_sc as plsc

VOCAB = 100000
EMBED = 64
BATCH = 4096
SEQ = 50
TOTAL = BATCH * SEQ
HALF = BATCH // 2

SPLIT = 51200
PACKED = SPLIT

NC = 2
NS = 16
NW = NC * NS
RPW = TOTAL // NW
CHUNK = 128
NCH = RPW // CHUNK
K = 5
G = NCH // K
NPLANE = 3

TV = 2048
OS = 10

_mesh = plsc.VectorSubcoreMesh(core_axis_name="c", subcore_axis_name="s")


@functools.partial(
    pl.kernel,
    out_type=jax.ShapeDtypeStruct((TOTAL, EMBED), jnp.float32),
    mesh=_mesh,
    compiler_params=pltpu.CompilerParams(use_tc_tiling_on_sc=False,
                                         needs_layout_passes=False),
    scratch_types=[
        pltpu.VMEM((NPLANE * BATCH,), jnp.int32),
        pltpu.VMEM((NCH, CHUNK), jnp.int32),
        pltpu.VMEM((K * CHUNK, EMBED), jnp.float32),
        pltpu.VMEM((K * CHUNK, EMBED), jnp.float32),
        pltpu.SemaphoreType.DMA,
        pltpu.SemaphoreType.DMA,
        pltpu.SemaphoreType.DMA,
        pltpu.SemaphoreType.DMA,
    ],
)
def _sc_gather(idx_hbm, tab_hbm, out_hbm,
               plane_v, idx_v, set_a, set_b, gs_a, gs_b, ws_a, ws_b):
    wid = lax.axis_index("s") * NC + lax.axis_index("c")
    base = wid * RPW
    s_lo = jnp.minimum(base // BATCH, SEQ - NPLANE)
    for k in range(NPLANE):
        pltpu.sync_copy(idx_hbm.at[s_lo + k],
                        plane_v.at[pl.ds(k * BATCH, BATCH)])

    def remap(a):
        a2 = a * 2
        return jnp.where(a < SPLIT, a2, a2 - (2 * SPLIT - 1))

    lanes = lax.iota(jnp.int32, 16)

    def build(c, carry):
        j0 = base + c * CHUNK
        s_off = j0 // BATCH - s_lo
        q0 = (j0 % BATCH) // 2
        rows = jnp.full((16,), c, jnp.int32)
        for i in range(4):
            off = s_off * BATCH + q0 + 16 * i
            a = plane_v[pl.ds(off, 16)]
            b = plane_v[pl.ds(off + HALF, 16)]
            cols = lanes * 2 + 32 * i
            plsc.store_scatter(idx_v, [rows, cols], remap(a))
            plsc.store_scatter(idx_v, [rows, cols + 1], remap(b))
        return carry

    lax.fori_loop(0, NCH, build, 0)

    def issue_gathers(g, dst, gsem):
        for i in range(K):
            pltpu.async_copy(tab_hbm.at[idx_v.at[g * K + i]],
                             dst.at[pl.ds(i * CHUNK, CHUNK)], gsem)

    def drain_gathers(dst, gsem):
        for i in range(K):
            pltpu.make_async_copy(
                tab_hbm.at[pl.ds(0, CHUNK)],
                dst.at[pl.ds(i * CHUNK, CHUNK)], gsem).wait()

    def issue_writeback(g, src, wsem):
        pltpu.async_copy(
            src, out_hbm.at[pl.ds(base + g * K * CHUNK, K * CHUNK)], wsem)

    def wait_writeback(src, wsem):
        pltpu.make_async_copy(
            src, out_hbm.at[pl.ds(base, K * CHUNK)], wsem).wait()

    issue_gathers(0, set_a, gs_a)

    def body(t, carry):
        @pl.when((t % 2) == 0)
        def _():
            @pl.when(t >= 1)
            def _():
                wait_writeback(set_b, ws_b)
            issue_gathers(t + 1, set_b, gs_b)
            drain_gathers(set_a, gs_a)
            issue_writeback(t, set_a, ws_a)

        @pl.when((t % 2) == 1)
        def _():
            wait_writeback(set_a, ws_a)
            issue_gathers(t + 1, set_a, gs_a)
            drain_gathers(set_b, gs_b)
            issue_writeback(t, set_b, ws_b)

        return carry

    lax.fori_loop(0, G - 1, body, 0)

    drain_gathers(set_b, gs_b)
    issue_writeback(G - 1, set_b, ws_b)
    wait_writeback(set_a, ws_a)
    wait_writeback(set_b, ws_b)


def _eye64():
    r = lax.broadcasted_iota(jnp.int32, (EMBED, EMBED), 0)
    c = lax.broadcasted_iota(jnp.int32, (EMBED, EMBED), 1)
    return (r == c).astype(jnp.float32)


def _tc_pack_body(inl_ref, inr_ref, out_ref):
    eye = _eye64()
    lo = lax.dot_general(inl_ref[...], eye, (((0,), (0,)), ((), ())),
                         preferred_element_type=jnp.float32)
    hi = lax.dot_general(inr_ref[...], eye, (((0,), (0,)), ((), ())),
                         preferred_element_type=jnp.float32)
    out_ref[...] = jnp.concatenate([lo, hi], axis=1)


_NB_L = SPLIT // TV


_tc_table_pack = pl.pallas_call(
    _tc_pack_body,
    grid=(_NB_L,),
    in_specs=[
        pl.BlockSpec((EMBED, TV), lambda i: (0, i)),
        pl.BlockSpec((EMBED, TV),
                     lambda i: (0, jnp.minimum(_NB_L + i,
                                               (VOCAB - 1) // TV))),
    ],
    out_specs=pl.BlockSpec((TV, 2 * EMBED), lambda i: (i, 0)),
    out_shape=jax.ShapeDtypeStruct((PACKED, 2 * EMBED), jnp.float32),
)


def _tc_out_body(in_ref, out_ref):
    eye = _eye64()
    for s in range(OS):
        blk = in_ref[pl.ds(s * HALF, HALF), :]
        lo = lax.dot_general(eye, blk[:, :EMBED],
                             (((1,), (1,)), ((), ())),
                             preferred_element_type=jnp.float32)
        hi = lax.dot_general(eye, blk[:, EMBED:],
                             (((1,), (1,)), ((), ())),
                             preferred_element_type=jnp.float32)
        out_ref[pl.ds(s * EMBED, EMBED), :] = jnp.concatenate(
            [lo, hi], axis=1)


_tc_out_transpose = pl.pallas_call(
    _tc_out_body,
    grid=(SEQ // OS,),
    in_specs=[pl.BlockSpec((OS * HALF, 2 * EMBED), lambda i: (i, 0))],
    out_specs=pl.BlockSpec((OS * EMBED, BATCH), lambda i: (i, 0)),
    out_shape=jax.ShapeDtypeStruct((SEQ * EMBED, BATCH), jnp.float32),
)


def kernel(src_indices, tgt_indices, source_table, target_table):
    st = _tc_table_pack(jnp.transpose(source_table),
                        jnp.transpose(source_table))
    tt = _tc_table_pack(jnp.transpose(target_table),
                        jnp.transpose(target_table))
    so = _sc_gather(jnp.transpose(src_indices), st.reshape(2 * PACKED, EMBED))
    to = _sc_gather(jnp.transpose(tgt_indices), tt.reshape(2 * PACKED, EMBED))

    def to_entry(y):
        z = _tc_out_transpose(y.reshape(TOTAL // 2, 2 * EMBED))
        return jnp.transpose(z.reshape(SEQ, EMBED, BATCH), (2, 0, 1))

    return (to_entry(so), to_entry(to))
